# baseline (speedup 1.0000x reference)
import functools

import jax
import jax.numpy as jnp
from jax import lax
from jax.experimental import pallas as pl
from jax.experimental.pallas import tpu as pltpu
from jax.experimental.pallas import tpu_sc as plsc

_N = 8388608
_NW = 32
_SC_PART = 3145728
_PER_W = _SC_PART // _NW
_SC_CHUNK = 32768
_NCHUNK = _PER_W // _SC_CHUNK
_CNT_BLK = 524288
_TC_OFF = _SC_PART // _CNT_BLK
_TC_GRID = (_N - _SC_PART) // _CNT_BLK
_OUT_BLK = 16384


def _sc_count(x_hbm, out_hbm, buf0, buf1, acc_v, sem0, sem1):
    wid = lax.axis_index("s") * 2 + lax.axis_index("c")
    base = wid * _PER_W
    bufs = (buf0, buf1)
    sems = (sem0, sem1)
    pending = [None, None]
    pending[0] = pltpu.async_copy(x_hbm.at[pl.ds(base, _SC_CHUNK)], buf0, sem0)
    accs = tuple(jnp.zeros((16,), jnp.int32) for _ in range(4))
    for c in range(_NCHUNK):
        if c + 1 < _NCHUNK:
            nb = (c + 1) & 1
            pending[nb] = pltpu.async_copy(
                x_hbm.at[pl.ds(base + (c + 1) * _SC_CHUNK, _SC_CHUNK)],
                bufs[nb], sems[nb])
        pending[c & 1].wait()
        buf = bufs[c & 1]

        def body(i, a, buf=buf):
            a0, a1, a2, a3 = a
            b = i * 256
            for j in range(4):
                o = b + j * 64
                a0 = a0 + buf[pl.ds(o, 16)]
                a1 = a1 + buf[pl.ds(o + 16, 16)]
                a2 = a2 + buf[pl.ds(o + 32, 16)]
                a3 = a3 + buf[pl.ds(o + 48, 16)]
            return (a0, a1, a2, a3)

        accs = lax.fori_loop(0, _SC_CHUNK // 256, body, accs)
    acc_v[...] = accs[0] + accs[1] + accs[2] + accs[3]
    pltpu.sync_copy(acc_v, out_hbm.at[pl.ds(wid * 16, 16)])


_sc_count_call = functools.partial(
    pl.kernel,
    out_type=jax.ShapeDtypeStruct((_NW * 16,), jnp.int32),
    mesh=plsc.VectorSubcoreMesh(core_axis_name="c", subcore_axis_name="s"),
    scratch_types=[
        pltpu.VMEM((_SC_CHUNK,), jnp.int32),
        pltpu.VMEM((_SC_CHUNK,), jnp.int32),
        pltpu.VMEM((16,), jnp.int32),
        pltpu.SemaphoreType.DMA,
        pltpu.SemaphoreType.DMA,
    ],
)(_sc_count)


def _tc_count_kernel(x_ref, tsum_ref, acc_ref):
    i = pl.program_id(0)

    @pl.when(i == 0)
    def _():
        acc_ref[0] = 0

    acc_ref[0] += jnp.sum(x_ref[...])

    @pl.when(i == pl.num_programs(0) - 1)
    def _():
        tsum_ref[0] = acc_ref[0]


def _bcast_kernel(p_ref, t_ref, o_ref, maj_ref):
    @pl.when(pl.program_id(0) == 0)
    def _():
        total = jnp.sum(p_ref[...]) + t_ref[0]
        maj_ref[0] = (2 * total > _N).astype(jnp.int32)

    c = lax.broadcasted_iota(jnp.int32, o_ref.shape, 1)
    o_ref[...] = (c == maj_ref[0]).astype(jnp.float32)


def kernel(x):
    partials = _sc_count_call(x)

    tsum = pl.pallas_call(
        _tc_count_kernel,
        grid=(_TC_GRID,),
        in_specs=[pl.BlockSpec((_CNT_BLK,), lambda i: (i + _TC_OFF,))],
        out_specs=pl.BlockSpec(memory_space=pltpu.SMEM),
        out_shape=jax.ShapeDtypeStruct((1,), jnp.int32),
        scratch_shapes=[pltpu.SMEM((1,), jnp.int32)],
    )(x)

    groups = _N // 128
    pred = pl.pallas_call(
        _bcast_kernel,
        grid=(groups // _OUT_BLK,),
        in_specs=[
            pl.BlockSpec((_NW * 16,), lambda i: (0,)),
            pl.BlockSpec(memory_space=pltpu.SMEM),
        ],
        out_specs=pl.BlockSpec((_OUT_BLK, 2, 128), lambda i: (i, 0, 0)),
        out_shape=jax.ShapeDtypeStruct((groups, 2, 128), jnp.float32),
        scratch_shapes=[pltpu.SMEM((1,), jnp.int32)],
    )(partials, tsum)

    return pred.transpose(0, 2, 1).reshape(_N, 2)


# speedup vs baseline: 1.0288x; 1.0288x over previous
import functools

import jax
import jax.numpy as jnp
from jax import lax
from jax.experimental import pallas as pl
from jax.experimental.pallas import tpu as pltpu
from jax.experimental.pallas import tpu_sc as plsc

_N = 8388608
_NW = 32
_SC_PART = 3145728
_PER_W = _SC_PART // _NW
_SC_CHUNK = 32768
_NCHUNK = _PER_W // _SC_CHUNK
_CNT_BLK = 524288
_TC_OFF = _SC_PART // _CNT_BLK
_TC_GRID = (_N - _SC_PART) // _CNT_BLK
_OUT_BLK = 8192


def _sc_count(x_hbm, out_hbm, buf0, buf1, acc_v, sem0, sem1):
    wid = lax.axis_index("s") * 2 + lax.axis_index("c")
    base = wid * _PER_W
    bufs = (buf0, buf1)
    sems = (sem0, sem1)
    pending = [None, None]
    pending[0] = pltpu.async_copy(x_hbm.at[pl.ds(base, _SC_CHUNK)], buf0, sem0)
    accs = tuple(jnp.zeros((16,), jnp.int32) for _ in range(4))
    for c in range(_NCHUNK):
        if c + 1 < _NCHUNK:
            nb = (c + 1) & 1
            pending[nb] = pltpu.async_copy(
                x_hbm.at[pl.ds(base + (c + 1) * _SC_CHUNK, _SC_CHUNK)],
                bufs[nb], sems[nb])
        pending[c & 1].wait()
        buf = bufs[c & 1]

        def body(i, a, buf=buf):
            a0, a1, a2, a3 = a
            b = i * 256
            for j in range(4):
                o = b + j * 64
                a0 = a0 + buf[pl.ds(o, 16)]
                a1 = a1 + buf[pl.ds(o + 16, 16)]
                a2 = a2 + buf[pl.ds(o + 32, 16)]
                a3 = a3 + buf[pl.ds(o + 48, 16)]
            return (a0, a1, a2, a3)

        accs = lax.fori_loop(0, _SC_CHUNK // 256, body, accs)
    acc_v[...] = accs[0] + accs[1] + accs[2] + accs[3]
    pltpu.sync_copy(acc_v, out_hbm.at[pl.ds(wid * 16, 16)])


_sc_count_call = functools.partial(
    pl.kernel,
    out_type=jax.ShapeDtypeStruct((_NW * 16,), jnp.int32),
    mesh=plsc.VectorSubcoreMesh(core_axis_name="c", subcore_axis_name="s"),
    scratch_types=[
        pltpu.VMEM((_SC_CHUNK,), jnp.int32),
        pltpu.VMEM((_SC_CHUNK,), jnp.int32),
        pltpu.VMEM((16,), jnp.int32),
        pltpu.SemaphoreType.DMA,
        pltpu.SemaphoreType.DMA,
    ],
)(_sc_count)


def _tc_count_kernel(x_ref, tsum_ref, acc_ref):
    i = pl.program_id(0)

    @pl.when(i == 0)
    def _():
        acc_ref[0] = 0

    acc_ref[0] += jnp.sum(x_ref[...])

    @pl.when(i == pl.num_programs(0) - 1)
    def _():
        tsum_ref[0] = acc_ref[0]


def _bcast_kernel(p_ref, t_ref, o_ref, maj_ref):
    @pl.when(pl.program_id(0) == 0)
    def _():
        total = jnp.sum(p_ref[...]) + t_ref[0]
        maj_ref[0] = (2 * total > _N).astype(jnp.int32)

    c = lax.broadcasted_iota(jnp.int32, o_ref.shape, 1)
    o_ref[...] = (c == maj_ref[0]).astype(jnp.float32)


def kernel(x):
    partials = _sc_count_call(x)

    tsum = pl.pallas_call(
        _tc_count_kernel,
        grid=(_TC_GRID,),
        in_specs=[pl.BlockSpec((_CNT_BLK,), lambda i: (i + _TC_OFF,))],
        out_specs=pl.BlockSpec(memory_space=pltpu.SMEM),
        out_shape=jax.ShapeDtypeStruct((1,), jnp.int32),
        scratch_shapes=[pltpu.SMEM((1,), jnp.int32)],
    )(x)

    groups = _N // 128
    pred = pl.pallas_call(
        _bcast_kernel,
        grid=(groups // _OUT_BLK,),
        in_specs=[
            pl.BlockSpec((_NW * 16,), lambda i: (0,)),
            pl.BlockSpec(memory_space=pltpu.SMEM),
        ],
        out_specs=pl.BlockSpec((_OUT_BLK, 2, 128), lambda i: (i, 0, 0)),
        out_shape=jax.ShapeDtypeStruct((groups, 2, 128), jnp.float32),
        scratch_shapes=[pltpu.SMEM((1,), jnp.int32)],
    )(partials, tsum)

    return pred.transpose(0, 2, 1).reshape(_N, 2)
